# Initial kernel scaffold; baseline (speedup 1.0000x reference)
import functools

import jax
import jax.numpy as jnp
from jax import lax
from jax.experimental import pallas as pl
from jax.experimental.pallas import tpu as pltpu
from jax.experimental.pallas import tpu_sc as plsc

NUM_USERS = 25000
NUM_ITEMS = 25000
N = NUM_USERS + NUM_ITEMS
E = 1600000
D = 32
N_LAYERS = 3

NC, NS = 2, 16
NT = NC * NS
N_PAD = 50176
R_T = N_PAD // NS
T_E = 50176
E_PAD = T_E * NT
CHUNK = 1024
UNITS = CHUNK // 128
N_CHUNKS = T_E // CHUNK
TC_BLK = 512

_mesh = plsc.VectorSubcoreMesh(
    core_axis_name="c", subcore_axis_name="s", num_cores=NC, num_subcores=NS)


@functools.partial(
    pl.kernel,
    out_type=jax.ShapeDtypeStruct((NC, N_PAD, D), jnp.float32),
    mesh=_mesh,
    scratch_types=[
        pltpu.VMEM((UNITS, 128), jnp.int32),
        pltpu.VMEM((UNITS, 128), jnp.int32),
        pltpu.VMEM((128, D), jnp.float32),
        pltpu.VMEM((128, D), jnp.float32),
        pltpu.VMEM_SHARED((N_PAD, D), jnp.float32),
        pltpu.SemaphoreType.DMA,
        pltpu.SemaphoreType.DMA,
        pltpu.SemaphoreType.DMA,
        pltpu.SemaphoreType.DMA,
    ],
)
def _sc_segment_sum(z_hbm, src_hbm, dst_hbm, zeros_hbm, out_hbm,
                    sidx, didx, rows_a, rows_b, acc, gs_a, gs_b, ss_a, ss_b):
    c = lax.axis_index("c")
    s = lax.axis_index("s")
    wid = s * NC + c
    rbase = s * R_T
    pltpu.sync_copy(zeros_hbm.at[pl.ds(rbase, R_T)], acc.at[pl.ds(rbase, R_T)])
    plsc.subcore_barrier()

    cbase = wid * (T_E // 128)
    bufs = (rows_a, rows_b)
    gsems = (gs_a, gs_b)
    ssems = (ss_a, ss_b)

    @pl.loop(0, N_CHUNKS)
    def _(k):
        crow = cbase + k * UNITS
        pltpu.sync_copy(src_hbm.at[pl.ds(crow, UNITS)], sidx)
        pltpu.sync_copy(dst_hbm.at[pl.ds(crow, UNITS)], didx)
        g = [None] * UNITS
        sc = [None] * UNITS
        g[0] = pltpu.async_copy(z_hbm.at[sidx.at[0]], rows_a, gs_a)
        for j in range(UNITS):
            b = j % 2
            g[j].wait()
            if j + 1 < UNITS:
                if j >= 1:
                    sc[j - 1].wait()
                g[j + 1] = pltpu.async_copy(
                    z_hbm.at[sidx.at[j + 1]], bufs[1 - b], gsems[1 - b])
            sc[j] = pltpu.async_copy(
                bufs[b], acc.at[didx.at[j]], ssems[b], add=True)
        sc[UNITS - 2].wait()
        sc[UNITS - 1].wait()

    plsc.subcore_barrier()
    pltpu.sync_copy(acc.at[pl.ds(rbase, R_T)],
                    out_hbm.at[c, pl.ds(rbase, R_T)])


def _tc_call(body, n_in, n_out):
    spec = pl.BlockSpec((TC_BLK, D), lambda i: (i, 0))
    return pl.pallas_call(
        body,
        grid=(N_PAD // TC_BLK,),
        in_specs=[spec] * n_in,
        out_specs=[spec] * n_out if n_out > 1 else spec,
        out_shape=(
            [jax.ShapeDtypeStruct((N_PAD, D), jnp.float32)] * n_out
            if n_out > 1 else jax.ShapeDtypeStruct((N_PAD, D), jnp.float32)),
    )


def _tc_init_body(d0_ref, d1_ref, x0_ref, dinv_ref, z0_ref):
    deg = d0_ref[...] + d1_ref[...]
    dinv = jnp.where(deg > 0, lax.rsqrt(jnp.maximum(deg, 1e-12)),
                     jnp.float32(0.0))
    dinv_ref[...] = dinv
    z0_ref[...] = x0_ref[...] * dinv


def _tc_combine_body(p0_ref, p1_ref, dinv_ref, x_ref, z_ref):
    dinv = dinv_ref[...]
    x = dinv * (p0_ref[...] + p1_ref[...])
    x_ref[...] = x
    z_ref[...] = x * dinv


def _tc_final_body(p0_ref, p1_ref, dinv_ref, x0_ref, x1_ref, x2_ref, out_ref):
    x3 = dinv_ref[...] * (p0_ref[...] + p1_ref[...])
    out_ref[...] = (x0_ref[...] + x1_ref[...] + x2_ref[...] + x3) * 0.25


def kernel(user_emb, item_emb, edge_index):
    src = edge_index[0].astype(jnp.int32)
    dst = edge_index[1].astype(jnp.int32)
    pad_e = E_PAD - E
    src2d = jnp.concatenate([src, jnp.zeros((pad_e,), jnp.int32)]
                            ).reshape(E_PAD // 128, 128)
    dst2d = jnp.concatenate([dst, jnp.full((pad_e,), N, jnp.int32)]
                            ).reshape(E_PAD // 128, 128)

    x0 = jnp.concatenate(
        [user_emb, item_emb,
         jnp.zeros((N_PAD - N, D), jnp.float32)], axis=0)
    zeros32 = jnp.zeros((N_PAD, D), jnp.float32)
    ones32 = jnp.ones((N_PAD, D), jnp.float32)

    degp = _sc_segment_sum(ones32, src2d, dst2d, zeros32)
    dinv, z = _tc_call(_tc_init_body, 3, 2)(degp[0], degp[1], x0)

    xs = []
    for _ in range(N_LAYERS - 1):
        p = _sc_segment_sum(z, src2d, dst2d, zeros32)
        x, z = _tc_call(_tc_combine_body, 3, 2)(p[0], p[1], dinv)
        xs.append(x)
    p = _sc_segment_sum(z, src2d, dst2d, zeros32)
    out = _tc_call(_tc_final_body, 6, 1)(p[0], p[1], dinv, x0, xs[0], xs[1])

    return (out[:NUM_USERS], out[NUM_USERS:N])


# speedup vs baseline: 19.9010x; 19.9010x over previous
import functools

import jax
import jax.numpy as jnp
from jax import lax
from jax.experimental import pallas as pl
from jax.experimental.pallas import tpu as pltpu
from jax.experimental.pallas import tpu_sc as plsc

NUM_USERS = 25000
NUM_ITEMS = 25000
N = NUM_USERS + NUM_ITEMS
E = 1600000
D = 32
N_LAYERS = 3

NC, NS = 2, 16
NT = NC * NS
N_PAD = 50176
R_T = N_PAD // NS
T_E = 50176
E_PAD = T_E * NT
CHUNK = 1024
UNITS = CHUNK // 128
N_CHUNKS = T_E // CHUNK
TC_BLK = 512

@functools.cache
def _make_sc_segment_sum():
  mesh = plsc.VectorSubcoreMesh(
      core_axis_name="c", subcore_axis_name="s", num_cores=NC, num_subcores=NS)

  @functools.partial(
      pl.kernel,
      out_type=jax.ShapeDtypeStruct((NC, N_PAD, D), jnp.float32),
      mesh=mesh,
      scratch_types=[
          pltpu.VMEM((UNITS, 128), jnp.int32),
          pltpu.VMEM((UNITS, 128), jnp.int32),
          pltpu.VMEM((128, D), jnp.float32),
          pltpu.VMEM((128, D), jnp.float32),
          pltpu.VMEM_SHARED((N_PAD, D), jnp.float32),
          pltpu.SemaphoreType.DMA,
          pltpu.SemaphoreType.DMA,
          pltpu.SemaphoreType.DMA,
          pltpu.SemaphoreType.DMA,
      ],
      compiler_params=pltpu.CompilerParams(use_tc_tiling_on_sc=False),
  )
  def sc_segment_sum(z_hbm, src_hbm, dst_hbm, zeros_hbm, out_hbm,
                     sidx, didx, rows_a, rows_b, acc, gs_a, gs_b, ss_a, ss_b):
    c = lax.axis_index("c")
    s = lax.axis_index("s")
    wid = s * NC + c
    rbase = s * R_T
    pltpu.sync_copy(zeros_hbm.at[pl.ds(rbase, R_T)], acc.at[pl.ds(rbase, R_T)])
    plsc.subcore_barrier()

    cbase = wid * (T_E // 128)
    bufs = (rows_a, rows_b)
    gsems = (gs_a, gs_b)
    ssems = (ss_a, ss_b)

    @pl.loop(0, N_CHUNKS)
    def _(k):
      crow = cbase + k * UNITS
      pltpu.sync_copy(src_hbm.at[pl.ds(crow, UNITS)], sidx)
      pltpu.sync_copy(dst_hbm.at[pl.ds(crow, UNITS)], didx)
      g = [None] * UNITS
      sc = [None] * UNITS
      g[0] = pltpu.async_copy(z_hbm.at[sidx.at[0]], rows_a, gs_a)
      for j in range(UNITS):
        b = j % 2
        g[j].wait()
        if j + 1 < UNITS:
          if j >= 1:
            sc[j - 1].wait()
          g[j + 1] = pltpu.async_copy(
              z_hbm.at[sidx.at[j + 1]], bufs[1 - b], gsems[1 - b])
        sc[j] = pltpu.async_copy(
            bufs[b], acc.at[didx.at[j]], ssems[b], add=True)
      sc[UNITS - 2].wait()
      sc[UNITS - 1].wait()

    plsc.subcore_barrier()
    pltpu.sync_copy(acc.at[pl.ds(rbase, R_T)],
                    out_hbm.at[c, pl.ds(rbase, R_T)])

  return sc_segment_sum


def _tc_call(body, n_in, n_out):
    spec = pl.BlockSpec((TC_BLK, D), lambda i: (i, 0))
    return pl.pallas_call(
        body,
        grid=(N_PAD // TC_BLK,),
        in_specs=[spec] * n_in,
        out_specs=[spec] * n_out if n_out > 1 else spec,
        out_shape=(
            [jax.ShapeDtypeStruct((N_PAD, D), jnp.float32)] * n_out
            if n_out > 1 else jax.ShapeDtypeStruct((N_PAD, D), jnp.float32)),
    )


def _tc_init_body(d0_ref, d1_ref, x0_ref, dinv_ref, z0_ref):
    deg = d0_ref[...] + d1_ref[...]
    dinv = jnp.where(deg > 0, lax.rsqrt(jnp.maximum(deg, 1e-12)),
                     jnp.float32(0.0))
    dinv_ref[...] = dinv
    z0_ref[...] = x0_ref[...] * dinv


def _tc_combine_body(p0_ref, p1_ref, dinv_ref, x_ref, z_ref):
    dinv = dinv_ref[...]
    x = dinv * (p0_ref[...] + p1_ref[...])
    x_ref[...] = x
    z_ref[...] = x * dinv


def _tc_final_body(p0_ref, p1_ref, dinv_ref, x0_ref, x1_ref, x2_ref, out_ref):
    x3 = dinv_ref[...] * (p0_ref[...] + p1_ref[...])
    out_ref[...] = (x0_ref[...] + x1_ref[...] + x2_ref[...] + x3) * 0.25


def kernel(user_emb, item_emb, edge_index):
    src = edge_index[0].astype(jnp.int32)
    dst = edge_index[1].astype(jnp.int32)
    pad_e = E_PAD - E
    src2d = jnp.concatenate([src, jnp.zeros((pad_e,), jnp.int32)]
                            ).reshape(E_PAD // 128, 128)
    dst2d = jnp.concatenate([dst, jnp.full((pad_e,), N, jnp.int32)]
                            ).reshape(E_PAD // 128, 128)

    x0 = jnp.concatenate(
        [user_emb, item_emb,
         jnp.zeros((N_PAD - N, D), jnp.float32)], axis=0)
    zeros32 = jnp.zeros((N_PAD, D), jnp.float32)
    ones32 = jnp.ones((N_PAD, D), jnp.float32)

    sc_segment_sum = _make_sc_segment_sum()
    degp = sc_segment_sum(ones32, src2d, dst2d, zeros32)
    dinv, z = _tc_call(_tc_init_body, 3, 2)(degp[0], degp[1], x0)

    xs = []
    for _ in range(N_LAYERS - 1):
        p = sc_segment_sum(z, src2d, dst2d, zeros32)
        x, z = _tc_call(_tc_combine_body, 3, 2)(p[0], p[1], dinv)
        xs.append(x)
    p = sc_segment_sum(z, src2d, dst2d, zeros32)
    out = _tc_call(_tc_final_body, 6, 1)(p[0], p[1], dinv, x0, xs[0], xs[1])

    return (out[:NUM_USERS], out[NUM_USERS:N])


# speedup vs baseline: 30.6837x; 1.5418x over previous
import functools

import jax
import jax.numpy as jnp
from jax import lax
from jax.experimental import pallas as pl
from jax.experimental.pallas import tpu as pltpu
from jax.experimental.pallas import tpu_sc as plsc

NUM_USERS = 25000
NUM_ITEMS = 25000
N = NUM_USERS + NUM_ITEMS
E = 1600000
D = 32
N_LAYERS = 3

NC, NS = 2, 16
NT = NC * NS
N_PAD = 50176
R_T = N_PAD // NS
CHUNK = 768
UNITS = CHUNK // 128
N_CHUNKS = 66
T_E = N_CHUNKS * CHUNK
E_PAD = T_E * NT
TC_BLK = 512
D_DEG = 16

def _sc_compiler_params():
  return pltpu.CompilerParams(use_tc_tiling_on_sc=False)


@functools.cache
def _make_sc_segment_sum():
  mesh = plsc.VectorSubcoreMesh(
      core_axis_name="c", subcore_axis_name="s", num_cores=NC, num_subcores=NS)

  @functools.partial(
      pl.kernel,
      out_type=jax.ShapeDtypeStruct((NC, N_PAD, D), jnp.float32),
      mesh=mesh,
      scratch_types=[
          pltpu.VMEM((UNITS, 128), jnp.int32),
          pltpu.VMEM((UNITS, 128), jnp.int32),
          pltpu.VMEM((CHUNK, D), jnp.float32),
          pltpu.VMEM_SHARED((N_PAD, D), jnp.float32),
          pltpu.SemaphoreType.DMA,
          pltpu.SemaphoreType.DMA,
      ],
      compiler_params=_sc_compiler_params(),
  )
  def sc_segment_sum(z_hbm, src_hbm, dst_hbm, zeros_hbm, out_hbm,
                     sidx, didx, rows, acc, gsem, ssem):
    c = lax.axis_index("c")
    s = lax.axis_index("s")
    wid = s * NC + c
    rbase = s * R_T
    pltpu.sync_copy(zeros_hbm.at[pl.ds(rbase, R_T)], acc.at[pl.ds(rbase, R_T)])
    plsc.subcore_barrier()

    cbase = wid * (T_E // 128)

    @pl.loop(0, N_CHUNKS)
    def _(k):
      crow = cbase + k * UNITS
      pltpu.sync_copy(src_hbm.at[pl.ds(crow, UNITS)], sidx)
      pltpu.sync_copy(dst_hbm.at[pl.ds(crow, UNITS)], didx)
      g = [pltpu.async_copy(z_hbm.at[sidx.at[j]],
                            rows.at[pl.ds(j * 128, 128)], gsem)
           for j in range(UNITS)]
      for d in g:
        d.wait()
      sc = [pltpu.async_copy(rows.at[pl.ds(j * 128, 128)],
                             acc.at[didx.at[j]], ssem, add=True)
            for j in range(UNITS)]
      for d in sc:
        d.wait()

    plsc.subcore_barrier()
    pltpu.sync_copy(acc.at[pl.ds(rbase, R_T)],
                    out_hbm.at[c, pl.ds(rbase, R_T)])

  return sc_segment_sum


@functools.cache
def _make_sc_degree():
  mesh = plsc.VectorSubcoreMesh(
      core_axis_name="c", subcore_axis_name="s", num_cores=NC, num_subcores=NS)

  @functools.partial(
      pl.kernel,
      out_type=jax.ShapeDtypeStruct((NC, N_PAD, D_DEG), jnp.float32),
      mesh=mesh,
      scratch_types=[
          pltpu.VMEM((UNITS, 128), jnp.int32),
          pltpu.VMEM((128, D_DEG), jnp.float32),
          pltpu.VMEM_SHARED((N_PAD, D_DEG), jnp.float32),
          pltpu.SemaphoreType.DMA,
      ],
      compiler_params=_sc_compiler_params(),
  )
  def sc_degree(ones_hbm, dst_hbm, zeros_hbm, out_hbm,
                didx, ones_rows, acc, ssem):
    c = lax.axis_index("c")
    s = lax.axis_index("s")
    wid = s * NC + c
    rbase = s * R_T
    pltpu.sync_copy(zeros_hbm.at[pl.ds(rbase, R_T)], acc.at[pl.ds(rbase, R_T)])
    pltpu.sync_copy(ones_hbm.at[pl.ds(0, 128)], ones_rows)
    plsc.subcore_barrier()

    cbase = wid * (T_E // 128)

    @pl.loop(0, N_CHUNKS)
    def _(k):
      crow = cbase + k * UNITS
      pltpu.sync_copy(dst_hbm.at[pl.ds(crow, UNITS)], didx)
      sc = [pltpu.async_copy(ones_rows, acc.at[didx.at[j]], ssem, add=True)
            for j in range(UNITS)]
      for d in sc:
        d.wait()

    plsc.subcore_barrier()
    pltpu.sync_copy(acc.at[pl.ds(rbase, R_T)],
                    out_hbm.at[c, pl.ds(rbase, R_T)])

  return sc_degree


def _tc_call(body, n_in, n_out, in_widths=None):
    spec = pl.BlockSpec((TC_BLK, D), lambda i: (i, 0))
    if in_widths is None:
        in_specs = [spec] * n_in
    else:
        in_specs = [pl.BlockSpec((TC_BLK, w), lambda i: (i, 0))
                    for w in in_widths]
    return pl.pallas_call(
        body,
        grid=(N_PAD // TC_BLK,),
        in_specs=in_specs,
        out_specs=[spec] * n_out if n_out > 1 else spec,
        out_shape=(
            [jax.ShapeDtypeStruct((N_PAD, D), jnp.float32)] * n_out
            if n_out > 1 else jax.ShapeDtypeStruct((N_PAD, D), jnp.float32)),
    )


def _tc_init_body(d0_ref, d1_ref, x0_ref, dinv_ref, z0_ref):
    deg = d0_ref[...] + d1_ref[...]
    dinv16 = jnp.where(deg > 0, lax.rsqrt(jnp.maximum(deg, 1e-12)),
                       jnp.float32(0.0))
    dinv = jnp.concatenate([dinv16, dinv16], axis=1)
    dinv_ref[...] = dinv
    z0_ref[...] = x0_ref[...] * dinv


def _tc_combine_body(p0_ref, p1_ref, dinv_ref, x_ref, z_ref):
    dinv = dinv_ref[...]
    x = dinv * (p0_ref[...] + p1_ref[...])
    x_ref[...] = x
    z_ref[...] = x * dinv


def _tc_final_body(p0_ref, p1_ref, dinv_ref, x0_ref, x1_ref, x2_ref, out_ref):
    x3 = dinv_ref[...] * (p0_ref[...] + p1_ref[...])
    out_ref[...] = (x0_ref[...] + x1_ref[...] + x2_ref[...] + x3) * 0.25


def kernel(user_emb, item_emb, edge_index):
    src = edge_index[0].astype(jnp.int32)
    dst = edge_index[1].astype(jnp.int32)
    pad_e = E_PAD - E
    pad_ar = jnp.arange(pad_e, dtype=jnp.int32)
    src2d = jnp.concatenate([src, pad_ar % N]).reshape(E_PAD // 128, 128)
    dst2d = jnp.concatenate([dst, N + pad_ar % (N_PAD - N)]
                            ).reshape(E_PAD // 128, 128)

    x0 = jnp.concatenate(
        [user_emb, item_emb,
         jnp.zeros((N_PAD - N, D), jnp.float32)], axis=0)
    zeros32 = jnp.zeros((N_PAD, D), jnp.float32)
    zeros16 = jnp.zeros((N_PAD, D_DEG), jnp.float32)
    ones16 = jnp.ones((N_PAD, D_DEG), jnp.float32)

    sc_segment_sum = _make_sc_segment_sum()
    degp = _make_sc_degree()(ones16, dst2d, zeros16)
    dinv, z = _tc_call(_tc_init_body, 3, 2,
                       in_widths=(D_DEG, D_DEG, D))(degp[0], degp[1], x0)

    xs = []
    for _ in range(N_LAYERS - 1):
        p = sc_segment_sum(z, src2d, dst2d, zeros32)
        x, z = _tc_call(_tc_combine_body, 3, 2)(p[0], p[1], dinv)
        xs.append(x)
    p = sc_segment_sum(z, src2d, dst2d, zeros32)
    out = _tc_call(_tc_final_body, 6, 1)(p[0], p[1], dinv, x0, xs[0], xs[1])

    return (out[:NUM_USERS], out[NUM_USERS:N])


# speedup vs baseline: 39.9772x; 1.3029x over previous
import functools

import jax
import jax.numpy as jnp
from jax import lax
from jax.experimental import pallas as pl
from jax.experimental.pallas import tpu as pltpu
from jax.experimental.pallas import tpu_sc as plsc

NUM_USERS = 25000
NUM_ITEMS = 25000
N = NUM_USERS + NUM_ITEMS
E = 1600000
D = 32
N_LAYERS = 3

NC, NS = 2, 16
NT = NC * NS
N_PAD = 50176
R_T = N_PAD // NS
CHUNK = 384
UNITS = CHUNK // 128
N_PAIRS = 66
T_E = N_PAIRS * 2 * CHUNK
E_PAD = T_E * NT
F = N_PAD * D // 128
TC_BLK = 896


def _sc_compiler_params():
  return pltpu.CompilerParams(use_tc_tiling_on_sc=False)


def _fill_rows(rows_ref, n_rows, value):
  vec = jnp.full((16,), value, jnp.float32)

  @pl.loop(0, n_rows)
  def _(r):
    rows_ref[r, pl.ds(0, 16)] = vec
    rows_ref[r, pl.ds(16, 16)] = vec


def _zero_acc_slab(rows_ref, acc, rbase, buf_rows):
  n_full = R_T // buf_rows
  rem = R_T - n_full * buf_rows

  @pl.loop(0, n_full)
  def _(i):
    pltpu.sync_copy(rows_ref, acc.at[pl.ds(rbase + i * buf_rows, buf_rows)])

  if rem:
    pltpu.sync_copy(rows_ref.at[pl.ds(0, rem)],
                    acc.at[pl.ds(rbase + n_full * buf_rows, rem)])


def _dump_slab(acc, rbase, c, out0, out1):
  @pl.when(c == 0)
  def _():
    pltpu.sync_copy(acc.at[pl.ds(rbase, R_T)], out0.at[pl.ds(rbase, R_T)])

  @pl.when(c == 1)
  def _():
    pltpu.sync_copy(acc.at[pl.ds(rbase, R_T)], out1.at[pl.ds(rbase, R_T)])


@functools.cache
def _make_sc_segment_sum():
  mesh = plsc.VectorSubcoreMesh(
      core_axis_name="c", subcore_axis_name="s", num_cores=NC, num_subcores=NS)
  part = jax.ShapeDtypeStruct((N_PAD, D), jnp.float32)

  @functools.partial(
      pl.kernel,
      out_type=[part, part],
      mesh=mesh,
      scratch_types=[
          pltpu.VMEM((UNITS, 128), jnp.int32),
          pltpu.VMEM((UNITS, 128), jnp.int32),
          pltpu.VMEM((UNITS, 128), jnp.int32),
          pltpu.VMEM((UNITS, 128), jnp.int32),
          pltpu.VMEM((CHUNK, D), jnp.float32),
          pltpu.VMEM((CHUNK, D), jnp.float32),
          pltpu.VMEM_SHARED((N_PAD, D), jnp.float32),
          pltpu.SemaphoreType.DMA,
          pltpu.SemaphoreType.DMA,
          pltpu.SemaphoreType.DMA,
      ],
      compiler_params=_sc_compiler_params(),
  )
  def sc_segment_sum(z_hbm, src_hbm, dst_hbm, out0, out1,
                     sidx_a, sidx_b, didx_a, didx_b, rows_a, rows_b, acc,
                     gsem, ssem_a, ssem_b):
    c = lax.axis_index("c")
    s = lax.axis_index("s")
    wid = s * NC + c
    rbase = s * R_T
    _fill_rows(rows_a, CHUNK, 0.0)
    _zero_acc_slab(rows_a, acc, rbase, CHUNK)
    plsc.subcore_barrier()

    cbase = wid * (T_E // 128)

    def unit_rows(rows, u):
      return rows.at[pl.ds(u * 128, 128)]

    def scatter_descs(rows, didx, ssem):
      return [pltpu.make_async_copy(unit_rows(rows, u), acc.at[didx.at[u]],
                                    ssem) for u in range(UNITS)]

    def half(crow, sidx, didx, rows, ssem, drain_other):
      pltpu.sync_copy(src_hbm.at[pl.ds(crow, UNITS)], sidx)
      pltpu.sync_copy(dst_hbm.at[pl.ds(crow, UNITS)], didx)
      g = [pltpu.async_copy(z_hbm.at[sidx.at[u]], unit_rows(rows, u), gsem)
           for u in range(UNITS)]
      if drain_other is not None:
        for d in drain_other:
          d.wait()
      for d in g:
        d.wait()
      for u in range(UNITS):
        pltpu.async_copy(unit_rows(rows, u), acc.at[didx.at[u]], ssem,
                         add=True)

    def pair(p, first):
      crow_a = cbase + p * 2 * UNITS
      half(crow_a, sidx_a, didx_a, rows_a, ssem_a,
           None if first else scatter_descs(rows_a, didx_a, ssem_a))
      half(crow_a + UNITS, sidx_b, didx_b, rows_b, ssem_b,
           None if first else scatter_descs(rows_b, didx_b, ssem_b))

    pair(0, True)

    @pl.loop(1, N_PAIRS)
    def _(p):
      pair(p, False)

    for d in scatter_descs(rows_a, didx_a, ssem_a):
      d.wait()
    for d in scatter_descs(rows_b, didx_b, ssem_b):
      d.wait()

    plsc.subcore_barrier()
    _dump_slab(acc, rbase, c, out0, out1)

  return sc_segment_sum


@functools.cache
def _make_sc_degree():
  mesh = plsc.VectorSubcoreMesh(
      core_axis_name="c", subcore_axis_name="s", num_cores=NC, num_subcores=NS)
  part = jax.ShapeDtypeStruct((N_PAD, D), jnp.float32)
  DUNITS = 6

  @functools.partial(
      pl.kernel,
      out_type=[part, part],
      mesh=mesh,
      scratch_types=[
          pltpu.VMEM((DUNITS, 128), jnp.int32),
          pltpu.VMEM((128, D), jnp.float32),
          pltpu.VMEM_SHARED((N_PAD, D), jnp.float32),
          pltpu.SemaphoreType.DMA,
      ],
      compiler_params=_sc_compiler_params(),
  )
  def sc_degree(dst_hbm, out0, out1, didx, ones_rows, acc, ssem):
    c = lax.axis_index("c")
    s = lax.axis_index("s")
    wid = s * NC + c
    rbase = s * R_T
    _fill_rows(ones_rows, 128, 0.0)
    _zero_acc_slab(ones_rows, acc, rbase, 128)
    _fill_rows(ones_rows, 128, 1.0)
    plsc.subcore_barrier()

    cbase = wid * (T_E // 128)

    @pl.loop(0, T_E // (DUNITS * 128))
    def _(k):
      crow = cbase + k * DUNITS
      pltpu.sync_copy(dst_hbm.at[pl.ds(crow, DUNITS)], didx)
      sc = [pltpu.async_copy(ones_rows, acc.at[didx.at[u]], ssem, add=True)
            for u in range(DUNITS)]
      for d in sc:
        d.wait()

    plsc.subcore_barrier()
    _dump_slab(acc, rbase, c, out0, out1)

  return sc_degree


def _tc_call(body, n_in, n_out):
    spec = pl.BlockSpec((TC_BLK, 128), lambda i: (i, 0))
    return pl.pallas_call(
        body,
        grid=(F // TC_BLK,),
        in_specs=[spec] * n_in,
        out_specs=[spec] * n_out if n_out > 1 else spec,
        out_shape=(
            [jax.ShapeDtypeStruct((F, 128), jnp.float32)] * n_out
            if n_out > 1 else jax.ShapeDtypeStruct((F, 128), jnp.float32)),
    )


def _tc_init_body(d0_ref, d1_ref, x0_ref, dinv_ref, z0_ref):
    deg = d0_ref[...] + d1_ref[...]
    dinv = jnp.where(deg > 0, lax.rsqrt(jnp.maximum(deg, 1e-12)),
                     jnp.float32(0.0))
    dinv_ref[...] = dinv
    z0_ref[...] = x0_ref[...] * dinv


def _tc_combine_body(p0_ref, p1_ref, dinv_ref, x_ref, z_ref):
    dinv = dinv_ref[...]
    x = dinv * (p0_ref[...] + p1_ref[...])
    x_ref[...] = x
    z_ref[...] = x * dinv


def _tc_final_body(p0_ref, p1_ref, dinv_ref, x0_ref, x1_ref, x2_ref, out_ref):
    x3 = dinv_ref[...] * (p0_ref[...] + p1_ref[...])
    out_ref[...] = (x0_ref[...] + x1_ref[...] + x2_ref[...] + x3) * 0.25


def _flat(a):
    return a.reshape(F, 128)


def kernel(user_emb, item_emb, edge_index):
    src = edge_index[0].astype(jnp.int32)
    dst = edge_index[1].astype(jnp.int32)
    pad_e = E_PAD - E
    pad_ar = jnp.arange(pad_e, dtype=jnp.int32)
    src2d = jnp.concatenate([src, pad_ar % N]).reshape(E_PAD // 128, 128)
    dst2d = jnp.concatenate([dst, N + pad_ar % (N_PAD - N)]
                            ).reshape(E_PAD // 128, 128)

    x0 = jnp.concatenate(
        [user_emb, item_emb,
         jnp.zeros((N_PAD - N, D), jnp.float32)], axis=0)
    x0f = _flat(x0)

    sc_segment_sum = _make_sc_segment_sum()
    dg0, dg1 = _make_sc_degree()(dst2d)
    dinvf, zf = _tc_call(_tc_init_body, 3, 2)(_flat(dg0), _flat(dg1), x0f)

    xfs = []
    for _ in range(N_LAYERS - 1):
        p0, p1 = sc_segment_sum(zf.reshape(N_PAD, D), src2d, dst2d)
        xf, zf = _tc_call(_tc_combine_body, 3, 2)(_flat(p0), _flat(p1), dinvf)
        xfs.append(xf)
    p0, p1 = sc_segment_sum(zf.reshape(N_PAD, D), src2d, dst2d)
    outf = _tc_call(_tc_final_body, 6, 1)(
        _flat(p0), _flat(p1), dinvf, x0f, xfs[0], xfs[1])

    out = outf.reshape(N_PAD, D)
    return (out[:NUM_USERS], out[NUM_USERS:N])


# speedup vs baseline: 43.8498x; 1.0969x over previous
import functools

import jax
import jax.numpy as jnp
from jax import lax
from jax.experimental import pallas as pl
from jax.experimental.pallas import tpu as pltpu
from jax.experimental.pallas import tpu_sc as plsc

NUM_USERS = 25000
NUM_ITEMS = 25000
N = NUM_USERS + NUM_ITEMS
E = 1600000
D = 32
N_LAYERS = 3

NC, NS = 2, 16
NT = NC * NS
N_PAD = 50176
R_T = N_PAD // NS
CHUNK = 448
N_PAIRS = 57
T_E = N_PAIRS * 2 * CHUNK
E_PAD = T_E * NT
F = N_PAD * D // 128
TC_BLK = 896


def _sc_compiler_params():
  return pltpu.CompilerParams(use_tc_tiling_on_sc=False)


def _fill_rows(rows_ref, n_rows, value):
  vec = jnp.full((16,), value, jnp.float32)

  @pl.loop(0, n_rows)
  def _(r):
    rows_ref[r, pl.ds(0, 16)] = vec
    rows_ref[r, pl.ds(16, 16)] = vec


def _zero_acc_slab(rows_ref, acc, rbase, buf_rows):
  n_full = R_T // buf_rows
  rem = R_T - n_full * buf_rows

  @pl.loop(0, n_full)
  def _(i):
    pltpu.sync_copy(rows_ref, acc.at[pl.ds(rbase + i * buf_rows, buf_rows)])

  if rem:
    pltpu.sync_copy(rows_ref.at[pl.ds(0, rem)],
                    acc.at[pl.ds(rbase + n_full * buf_rows, rem)])


def _dump_slab(acc, rbase, c, out0, out1):
  @pl.when(c == 0)
  def _():
    pltpu.sync_copy(acc.at[pl.ds(rbase, R_T)], out0.at[pl.ds(rbase, R_T)])

  @pl.when(c == 1)
  def _():
    pltpu.sync_copy(acc.at[pl.ds(rbase, R_T)], out1.at[pl.ds(rbase, R_T)])


@functools.cache
def _make_sc_segment_sum():
  mesh = plsc.VectorSubcoreMesh(
      core_axis_name="c", subcore_axis_name="s", num_cores=NC, num_subcores=NS)
  part = jax.ShapeDtypeStruct((N_PAD, D), jnp.float32)

  @functools.partial(
      pl.kernel,
      out_type=[part, part],
      mesh=mesh,
      scratch_types=[
          pltpu.VMEM((CHUNK,), jnp.int32),
          pltpu.VMEM((CHUNK,), jnp.int32),
          pltpu.VMEM((CHUNK,), jnp.int32),
          pltpu.VMEM((CHUNK,), jnp.int32),
          pltpu.VMEM((CHUNK, D), jnp.float32),
          pltpu.VMEM((CHUNK, D), jnp.float32),
          pltpu.VMEM_SHARED((N_PAD, D), jnp.float32),
          pltpu.SemaphoreType.DMA,
          pltpu.SemaphoreType.DMA,
          pltpu.SemaphoreType.DMA,
      ],
      compiler_params=_sc_compiler_params(),
  )
  def sc_segment_sum(z_hbm, src_hbm, dst_hbm, out0, out1,
                     sidx_a, sidx_b, didx_a, didx_b, rows_a, rows_b, acc,
                     gsem, ssem_a, ssem_b):
    c = lax.axis_index("c")
    s = lax.axis_index("s")
    wid = s * NC + c
    rbase = s * R_T
    _fill_rows(rows_a, CHUNK, 0.0)
    _zero_acc_slab(rows_a, acc, rbase, CHUNK)
    plsc.subcore_barrier()

    ebase = wid * T_E

    def scatter_desc(rows, didx, ssem):
      return pltpu.make_async_copy(rows, acc.at[didx], ssem)

    def half(eoff, sidx, didx, rows, ssem, first):
      if not first:
        scatter_desc(rows, didx, ssem).wait()
      pltpu.sync_copy(src_hbm.at[pl.ds(eoff, CHUNK)], sidx)
      pltpu.sync_copy(dst_hbm.at[pl.ds(eoff, CHUNK)], didx)
      pltpu.async_copy(z_hbm.at[sidx], rows, gsem).wait()
      pltpu.async_copy(rows, acc.at[didx], ssem, add=True)

    def pair(p, first):
      eoff = ebase + p * 2 * CHUNK
      half(eoff, sidx_a, didx_a, rows_a, ssem_a, first)
      half(eoff + CHUNK, sidx_b, didx_b, rows_b, ssem_b, first)

    pair(0, True)

    @pl.loop(1, N_PAIRS)
    def _(p):
      pair(p, False)

    scatter_desc(rows_a, didx_a, ssem_a).wait()
    scatter_desc(rows_b, didx_b, ssem_b).wait()

    plsc.subcore_barrier()
    _dump_slab(acc, rbase, c, out0, out1)

  return sc_segment_sum


@functools.cache
def _make_sc_degree():
  mesh = plsc.VectorSubcoreMesh(
      core_axis_name="c", subcore_axis_name="s", num_cores=NC, num_subcores=NS)
  part = jax.ShapeDtypeStruct((N_PAD, D), jnp.float32)

  @functools.partial(
      pl.kernel,
      out_type=[part, part],
      mesh=mesh,
      scratch_types=[
          pltpu.VMEM((CHUNK,), jnp.int32),
          pltpu.VMEM((CHUNK,), jnp.int32),
          pltpu.VMEM((CHUNK, D), jnp.float32),
          pltpu.VMEM_SHARED((N_PAD, D), jnp.float32),
          pltpu.SemaphoreType.DMA,
          pltpu.SemaphoreType.DMA,
      ],
      compiler_params=_sc_compiler_params(),
  )
  def sc_degree(dst_hbm, out0, out1, didx_a, didx_b, ones_rows, acc,
                ssem_a, ssem_b):
    c = lax.axis_index("c")
    s = lax.axis_index("s")
    wid = s * NC + c
    rbase = s * R_T
    _fill_rows(ones_rows, CHUNK, 0.0)
    _zero_acc_slab(ones_rows, acc, rbase, CHUNK)
    _fill_rows(ones_rows, CHUNK, 1.0)
    plsc.subcore_barrier()

    ebase = wid * T_E

    def half(eoff, didx, ssem, first):
      if not first:
        pltpu.make_async_copy(ones_rows, acc.at[didx], ssem).wait()
      pltpu.sync_copy(dst_hbm.at[pl.ds(eoff, CHUNK)], didx)
      pltpu.async_copy(ones_rows, acc.at[didx], ssem, add=True)

    def pair(p, first):
      eoff = ebase + p * 2 * CHUNK
      half(eoff, didx_a, ssem_a, first)
      half(eoff + CHUNK, didx_b, ssem_b, first)

    pair(0, True)

    @pl.loop(1, N_PAIRS)
    def _(p):
      pair(p, False)

    pltpu.make_async_copy(ones_rows, acc.at[didx_a], ssem_a).wait()
    pltpu.make_async_copy(ones_rows, acc.at[didx_b], ssem_b).wait()

    plsc.subcore_barrier()
    _dump_slab(acc, rbase, c, out0, out1)

  return sc_degree


def _tc_call(body, n_in, n_out):
    spec = pl.BlockSpec((TC_BLK, 128), lambda i: (i, 0))
    return pl.pallas_call(
        body,
        grid=(F // TC_BLK,),
        in_specs=[spec] * n_in,
        out_specs=[spec] * n_out if n_out > 1 else spec,
        out_shape=(
            [jax.ShapeDtypeStruct((F, 128), jnp.float32)] * n_out
            if n_out > 1 else jax.ShapeDtypeStruct((F, 128), jnp.float32)),
    )


def _tc_init_body(d0_ref, d1_ref, x0_ref, dinv_ref, z0_ref):
    deg = d0_ref[...] + d1_ref[...]
    dinv = jnp.where(deg > 0, lax.rsqrt(jnp.maximum(deg, 1e-12)),
                     jnp.float32(0.0))
    dinv_ref[...] = dinv
    z0_ref[...] = x0_ref[...] * dinv


def _tc_combine_body(p0_ref, p1_ref, dinv_ref, x_ref, z_ref):
    dinv = dinv_ref[...]
    x = dinv * (p0_ref[...] + p1_ref[...])
    x_ref[...] = x
    z_ref[...] = x * dinv


def _tc_final_body(p0_ref, p1_ref, dinv_ref, x0_ref, x1_ref, x2_ref, out_ref):
    x3 = dinv_ref[...] * (p0_ref[...] + p1_ref[...])
    out_ref[...] = (x0_ref[...] + x1_ref[...] + x2_ref[...] + x3) * 0.25


def _flat(a):
    return a.reshape(F, 128)


def kernel(user_emb, item_emb, edge_index):
    src = edge_index[0].astype(jnp.int32)
    dst = edge_index[1].astype(jnp.int32)
    pad_e = E_PAD - E
    pad_ar = jnp.arange(pad_e, dtype=jnp.int32)
    src1d = jnp.concatenate([src, pad_ar % N])
    dst1d = jnp.concatenate([dst, N + pad_ar % (N_PAD - N)])

    x0 = jnp.concatenate(
        [user_emb, item_emb,
         jnp.zeros((N_PAD - N, D), jnp.float32)], axis=0)
    x0f = _flat(x0)

    sc_segment_sum = _make_sc_segment_sum()
    dg0, dg1 = _make_sc_degree()(dst1d)
    dinvf, zf = _tc_call(_tc_init_body, 3, 2)(_flat(dg0), _flat(dg1), x0f)

    xfs = []
    for _ in range(N_LAYERS - 1):
        p0, p1 = sc_segment_sum(zf.reshape(N_PAD, D), src1d, dst1d)
        xf, zf = _tc_call(_tc_combine_body, 3, 2)(_flat(p0), _flat(p1), dinvf)
        xfs.append(xf)
    p0, p1 = sc_segment_sum(zf.reshape(N_PAD, D), src1d, dst1d)
    outf = _tc_call(_tc_final_body, 6, 1)(
        _flat(p0), _flat(p1), dinvf, x0f, xfs[0], xfs[1])

    out = outf.reshape(N_PAD, D)
    return (out[:NUM_USERS], out[NUM_USERS:N])


# speedup vs baseline: 55.8227x; 1.2730x over previous
import functools

import jax
import jax.numpy as jnp
from jax import lax
from jax.experimental import pallas as pl
from jax.experimental.pallas import tpu as pltpu
from jax.experimental.pallas import tpu_sc as plsc

NUM_USERS = 25000
NUM_ITEMS = 25000
N = NUM_USERS + NUM_ITEMS
E = 1600000
D = 32
N_LAYERS = 3

NC, NS = 2, 16
NT = NC * NS
N_PAD = 50176
R_T = N_PAD // NS
CHUNK = 440
T_E = E // NT
N_PAIRS = 56
TAIL = T_E - N_PAIRS * 2 * CHUNK - CHUNK
F = N_PAD * D // 128
TC_BLK = 896


def _sc_compiler_params():
  return pltpu.CompilerParams(use_tc_tiling_on_sc=False)


def _fill_rows(rows_ref, n_rows, value):
  vec = jnp.full((16,), value, jnp.float32)

  @pl.loop(0, n_rows)
  def _(r):
    rows_ref[r, pl.ds(0, 16)] = vec
    rows_ref[r, pl.ds(16, 16)] = vec


def _zero_acc_slab(rows_ref, acc, rbase, buf_rows):
  n_full = R_T // buf_rows
  rem = R_T - n_full * buf_rows

  @pl.loop(0, n_full)
  def _(i):
    pltpu.sync_copy(rows_ref, acc.at[pl.ds(rbase + i * buf_rows, buf_rows)])

  if rem:
    pltpu.sync_copy(rows_ref.at[pl.ds(0, rem)],
                    acc.at[pl.ds(rbase + n_full * buf_rows, rem)])


def _dump_slab(acc, rbase, c, out0, out1):
  @pl.when(c == 0)
  def _():
    pltpu.sync_copy(acc.at[pl.ds(rbase, R_T)], out0.at[pl.ds(rbase, R_T)])

  @pl.when(c == 1)
  def _():
    pltpu.sync_copy(acc.at[pl.ds(rbase, R_T)], out1.at[pl.ds(rbase, R_T)])


@functools.cache
def _make_sc_segment_sum():
  mesh = plsc.VectorSubcoreMesh(
      core_axis_name="c", subcore_axis_name="s", num_cores=NC, num_subcores=NS)
  part = jax.ShapeDtypeStruct((N_PAD, D), jnp.float32)

  @functools.partial(
      pl.kernel,
      out_type=[part, part],
      mesh=mesh,
      scratch_types=[
          pltpu.VMEM((CHUNK,), jnp.int32),
          pltpu.VMEM((CHUNK,), jnp.int32),
          pltpu.VMEM((CHUNK,), jnp.int32),
          pltpu.VMEM((CHUNK,), jnp.int32),
          pltpu.VMEM((TAIL,), jnp.int32),
          pltpu.VMEM((TAIL,), jnp.int32),
          pltpu.VMEM((CHUNK, D), jnp.float32),
          pltpu.VMEM((CHUNK, D), jnp.float32),
          pltpu.VMEM_SHARED((N_PAD, D), jnp.float32),
          pltpu.SemaphoreType.DMA,
          pltpu.SemaphoreType.DMA,
          pltpu.SemaphoreType.DMA,
      ],
      compiler_params=_sc_compiler_params(),
  )
  def sc_segment_sum(z_hbm, ei_hbm, out0, out1,
                     sidx_a, sidx_b, didx_a, didx_b, sidx_t, didx_t,
                     rows_a, rows_b, acc, gsem, ssem_a, ssem_b):
    c = lax.axis_index("c")
    s = lax.axis_index("s")
    wid = s * NC + c
    rbase = s * R_T
    _fill_rows(rows_a, CHUNK, 0.0)
    _zero_acc_slab(rows_a, acc, rbase, CHUNK)
    plsc.subcore_barrier()

    ebase = wid * T_E

    def scatter_desc(rows, didx, ssem):
      return pltpu.make_async_copy(rows, acc.at[didx], ssem)

    def load_idx(eoff, n, sidx, didx):
      pltpu.sync_copy(ei_hbm.at[0, pl.ds(eoff, n)], sidx)
      pltpu.sync_copy(ei_hbm.at[1, pl.ds(eoff, n)], didx)

    def pair(p, first):
      eoff = ebase + p * 2 * CHUNK
      if not first:
        scatter_desc(rows_a, didx_a, ssem_a).wait()
      load_idx(eoff, CHUNK, sidx_a, didx_a)
      ga = pltpu.async_copy(z_hbm.at[sidx_a], rows_a, gsem)
      if not first:
        scatter_desc(rows_b, didx_b, ssem_b).wait()
      load_idx(eoff + CHUNK, CHUNK, sidx_b, didx_b)
      gb = pltpu.async_copy(z_hbm.at[sidx_b], rows_b, gsem)
      ga.wait()
      pltpu.async_copy(rows_a, acc.at[didx_a], ssem_a, add=True)
      gb.wait()
      pltpu.async_copy(rows_b, acc.at[didx_b], ssem_b, add=True)

    pair(0, True)

    @pl.loop(1, N_PAIRS)
    def _(p):
      pair(p, False)

    eoff = ebase + N_PAIRS * 2 * CHUNK
    scatter_desc(rows_a, didx_a, ssem_a).wait()
    load_idx(eoff, CHUNK, sidx_a, didx_a)
    ga = pltpu.async_copy(z_hbm.at[sidx_a], rows_a, gsem)
    scatter_desc(rows_b, didx_b, ssem_b).wait()
    load_idx(eoff + CHUNK, TAIL, sidx_t, didx_t)
    rows_t = rows_b.at[pl.ds(0, TAIL)]
    gb = pltpu.async_copy(z_hbm.at[sidx_t], rows_t, gsem)
    ga.wait()
    pltpu.async_copy(rows_a, acc.at[didx_a], ssem_a, add=True)
    gb.wait()
    pltpu.async_copy(rows_t, acc.at[didx_t], ssem_b, add=True)
    scatter_desc(rows_a, didx_a, ssem_a).wait()
    pltpu.make_async_copy(rows_t, acc.at[didx_t], ssem_b).wait()

    plsc.subcore_barrier()
    _dump_slab(acc, rbase, c, out0, out1)

  return sc_segment_sum


@functools.cache
def _make_sc_degree():
  mesh = plsc.VectorSubcoreMesh(
      core_axis_name="c", subcore_axis_name="s", num_cores=NC, num_subcores=NS)
  part = jax.ShapeDtypeStruct((N_PAD, D), jnp.float32)

  @functools.partial(
      pl.kernel,
      out_type=[part, part],
      mesh=mesh,
      scratch_types=[
          pltpu.VMEM((CHUNK,), jnp.int32),
          pltpu.VMEM((CHUNK,), jnp.int32),
          pltpu.VMEM((TAIL,), jnp.int32),
          pltpu.VMEM((CHUNK, D), jnp.float32),
          pltpu.VMEM_SHARED((N_PAD, D), jnp.float32),
          pltpu.SemaphoreType.DMA,
          pltpu.SemaphoreType.DMA,
      ],
      compiler_params=_sc_compiler_params(),
  )
  def sc_degree(ei_hbm, out0, out1, didx_a, didx_b, didx_t, ones_rows, acc,
                ssem_a, ssem_b):
    c = lax.axis_index("c")
    s = lax.axis_index("s")
    wid = s * NC + c
    rbase = s * R_T
    _fill_rows(ones_rows, CHUNK, 0.0)
    _zero_acc_slab(ones_rows, acc, rbase, CHUNK)
    _fill_rows(ones_rows, CHUNK, 1.0)
    plsc.subcore_barrier()

    ebase = wid * T_E

    def half(eoff, didx, ssem, first):
      if not first:
        pltpu.make_async_copy(ones_rows, acc.at[didx], ssem).wait()
      pltpu.sync_copy(ei_hbm.at[1, pl.ds(eoff, CHUNK)], didx)
      pltpu.async_copy(ones_rows, acc.at[didx], ssem, add=True)

    def pair(p, first):
      eoff = ebase + p * 2 * CHUNK
      half(eoff, didx_a, ssem_a, first)
      half(eoff + CHUNK, didx_b, ssem_b, first)

    pair(0, True)

    @pl.loop(1, N_PAIRS)
    def _(p):
      pair(p, False)

    eoff = ebase + N_PAIRS * 2 * CHUNK
    half(eoff, didx_a, ssem_a, False)
    pltpu.sync_copy(ei_hbm.at[1, pl.ds(eoff + CHUNK, TAIL)], didx_t)
    ones_t = ones_rows.at[pl.ds(0, TAIL)]
    pltpu.make_async_copy(ones_rows, acc.at[didx_b], ssem_b).wait()
    pltpu.async_copy(ones_t, acc.at[didx_t], ssem_b, add=True)

    pltpu.make_async_copy(ones_rows, acc.at[didx_a], ssem_a).wait()
    pltpu.make_async_copy(ones_t, acc.at[didx_t], ssem_b).wait()

    plsc.subcore_barrier()
    _dump_slab(acc, rbase, c, out0, out1)

  return sc_degree


def _tc_call(body, n_in, n_out):
    spec = pl.BlockSpec((TC_BLK, 128), lambda i: (i, 0))
    return pl.pallas_call(
        body,
        grid=(F // TC_BLK,),
        in_specs=[spec] * n_in,
        out_specs=[spec] * n_out if n_out > 1 else spec,
        out_shape=(
            [jax.ShapeDtypeStruct((F, 128), jnp.float32)] * n_out
            if n_out > 1 else jax.ShapeDtypeStruct((F, 128), jnp.float32)),
    )


def _tc_init_body(d0_ref, d1_ref, x0_ref, dinv_ref, z0_ref):
    deg = d0_ref[...] + d1_ref[...]
    dinv = jnp.where(deg > 0, lax.rsqrt(jnp.maximum(deg, 1e-12)),
                     jnp.float32(0.0))
    dinv_ref[...] = dinv
    z0_ref[...] = x0_ref[...] * dinv


def _tc_combine_body(p0_ref, p1_ref, dinv_ref, x_ref, z_ref):
    dinv = dinv_ref[...]
    x = dinv * (p0_ref[...] + p1_ref[...])
    x_ref[...] = x
    z_ref[...] = x * dinv


def _tc_final_body(p0_ref, p1_ref, dinv_ref, x0_ref, x1_ref, x2_ref, out_ref):
    x3 = dinv_ref[...] * (p0_ref[...] + p1_ref[...])
    out_ref[...] = (x0_ref[...] + x1_ref[...] + x2_ref[...] + x3) * 0.25


def _flat(a):
    return a.reshape(F, 128)


def kernel(user_emb, item_emb, edge_index):
    ei = edge_index.astype(jnp.int32)

    x0 = jnp.concatenate(
        [user_emb, item_emb,
         jnp.zeros((N_PAD - N, D), jnp.float32)], axis=0)
    x0f = _flat(x0)

    sc_segment_sum = _make_sc_segment_sum()
    dg0, dg1 = _make_sc_degree()(ei)
    dinvf, zf = _tc_call(_tc_init_body, 3, 2)(_flat(dg0), _flat(dg1), x0f)

    xfs = []
    for _ in range(N_LAYERS - 1):
        p0, p1 = sc_segment_sum(zf.reshape(N_PAD, D), ei)
        xf, zf = _tc_call(_tc_combine_body, 3, 2)(_flat(p0), _flat(p1), dinvf)
        xfs.append(xf)
    p0, p1 = sc_segment_sum(zf.reshape(N_PAD, D), ei)
    outf = _tc_call(_tc_final_body, 6, 1)(
        _flat(p0), _flat(p1), dinvf, x0f, xfs[0], xfs[1])

    fu = NUM_USERS * D // 128
    users = outf[:fu].reshape(NUM_USERS, D)
    items = outf[fu:2 * fu].reshape(NUM_ITEMS, D)
    return (users, items)


# speedup vs baseline: 57.2922x; 1.0263x over previous
import functools

import jax
import jax.numpy as jnp
from jax import lax
from jax.experimental import pallas as pl
from jax.experimental.pallas import tpu as pltpu
from jax.experimental.pallas import tpu_sc as plsc

NUM_USERS = 25000
NUM_ITEMS = 25000
N = NUM_USERS + NUM_ITEMS
E = 1600000
D = 32
N_LAYERS = 3

NC, NS = 2, 16
NT = NC * NS
N_PAD = 50176
R_T = N_PAD // NS
CHUNK = 440
T_E = E // NT
N_PAIRS = 56
TAIL = T_E - N_PAIRS * 2 * CHUNK - CHUNK
F = N_PAD * D // 128
TC_BLK = 896


def _sc_compiler_params():
  return pltpu.CompilerParams(use_tc_tiling_on_sc=False)


def _fill_rows(rows_ref, n_rows, value):
  vec = jnp.full((16,), value, jnp.float32)

  @pl.loop(0, n_rows)
  def _(r):
    rows_ref[r, pl.ds(0, 16)] = vec
    rows_ref[r, pl.ds(16, 16)] = vec


def _zero_acc_slab(rows_ref, acc, rbase, buf_rows):
  n_full = R_T // buf_rows
  rem = R_T - n_full * buf_rows

  @pl.loop(0, n_full)
  def _(i):
    pltpu.sync_copy(rows_ref, acc.at[pl.ds(rbase + i * buf_rows, buf_rows)])

  if rem:
    pltpu.sync_copy(rows_ref.at[pl.ds(0, rem)],
                    acc.at[pl.ds(rbase + n_full * buf_rows, rem)])


def _dump_slab(acc, rbase, c, out0, out1):
  @pl.when(c == 0)
  def _():
    pltpu.sync_copy(acc.at[pl.ds(rbase, R_T)], out0.at[pl.ds(rbase, R_T)])

  @pl.when(c == 1)
  def _():
    pltpu.sync_copy(acc.at[pl.ds(rbase, R_T)], out1.at[pl.ds(rbase, R_T)])


@functools.cache
def _make_sc_segment_sum():
  mesh = plsc.VectorSubcoreMesh(
      core_axis_name="c", subcore_axis_name="s", num_cores=NC, num_subcores=NS)
  part = jax.ShapeDtypeStruct((N_PAD, D), jnp.float32)

  @functools.partial(
      pl.kernel,
      out_type=[part, part],
      mesh=mesh,
      scratch_types=[
          pltpu.VMEM((CHUNK,), jnp.int32),
          pltpu.VMEM((CHUNK,), jnp.int32),
          pltpu.VMEM((CHUNK,), jnp.int32),
          pltpu.VMEM((CHUNK,), jnp.int32),
          pltpu.VMEM((TAIL,), jnp.int32),
          pltpu.VMEM((TAIL,), jnp.int32),
          pltpu.VMEM((CHUNK, D), jnp.float32),
          pltpu.VMEM((CHUNK, D), jnp.float32),
          pltpu.VMEM_SHARED((N_PAD, D), jnp.float32),
          pltpu.SemaphoreType.DMA,
          pltpu.SemaphoreType.DMA,
          pltpu.SemaphoreType.DMA,
      ],
      compiler_params=_sc_compiler_params(),
  )
  def sc_segment_sum(z_hbm, ei_hbm, out0, out1,
                     sidx_a, sidx_b, didx_a, didx_b, sidx_t, didx_t,
                     rows_a, rows_b, acc, gsem, ssem_a, ssem_b):
    c = lax.axis_index("c")
    s = lax.axis_index("s")
    wid = s * NC + c
    rbase = s * R_T
    _fill_rows(rows_a, CHUNK, 0.0)
    _zero_acc_slab(rows_a, acc, rbase, CHUNK)
    plsc.subcore_barrier()

    ebase = wid * T_E

    def scatter_desc(rows, didx, ssem):
      return pltpu.make_async_copy(rows, acc.at[didx], ssem)

    def load_idx(eoff, n, sidx, didx):
      pltpu.sync_copy(ei_hbm.at[0, pl.ds(eoff, n)], sidx)
      pltpu.sync_copy(ei_hbm.at[1, pl.ds(eoff, n)], didx)

    def gather_split(sidx, rows):
      h1 = 224
      g1 = pltpu.async_copy(z_hbm.at[sidx.at[pl.ds(0, h1)]],
                            rows.at[pl.ds(0, h1)], gsem)
      g2 = pltpu.async_copy(z_hbm.at[sidx.at[pl.ds(h1, CHUNK - h1)]],
                            rows.at[pl.ds(h1, CHUNK - h1)], gsem)
      return (g1, g2)

    def pair(p, first):
      eoff = ebase + p * 2 * CHUNK
      if not first:
        scatter_desc(rows_a, didx_a, ssem_a).wait()
      load_idx(eoff, CHUNK, sidx_a, didx_a)
      ga = gather_split(sidx_a, rows_a)
      if not first:
        scatter_desc(rows_b, didx_b, ssem_b).wait()
      load_idx(eoff + CHUNK, CHUNK, sidx_b, didx_b)
      gb = gather_split(sidx_b, rows_b)
      for g in ga:
        g.wait()
      pltpu.async_copy(rows_a, acc.at[didx_a], ssem_a, add=True)
      for g in gb:
        g.wait()
      pltpu.async_copy(rows_b, acc.at[didx_b], ssem_b, add=True)

    pair(0, True)

    @pl.loop(1, N_PAIRS)
    def _(p):
      pair(p, False)

    eoff = ebase + N_PAIRS * 2 * CHUNK
    scatter_desc(rows_a, didx_a, ssem_a).wait()
    load_idx(eoff, CHUNK, sidx_a, didx_a)
    ga = gather_split(sidx_a, rows_a)
    scatter_desc(rows_b, didx_b, ssem_b).wait()
    load_idx(eoff + CHUNK, TAIL, sidx_t, didx_t)
    rows_t = rows_b.at[pl.ds(0, TAIL)]
    gb = pltpu.async_copy(z_hbm.at[sidx_t], rows_t, gsem)
    for g in ga:
      g.wait()
    pltpu.async_copy(rows_a, acc.at[didx_a], ssem_a, add=True)
    gb.wait()
    pltpu.async_copy(rows_t, acc.at[didx_t], ssem_b, add=True)
    scatter_desc(rows_a, didx_a, ssem_a).wait()
    pltpu.make_async_copy(rows_t, acc.at[didx_t], ssem_b).wait()

    plsc.subcore_barrier()
    _dump_slab(acc, rbase, c, out0, out1)

  return sc_segment_sum


@functools.cache
def _make_sc_degree():
  mesh = plsc.VectorSubcoreMesh(
      core_axis_name="c", subcore_axis_name="s", num_cores=NC, num_subcores=NS)
  part = jax.ShapeDtypeStruct((N_PAD, D), jnp.float32)

  @functools.partial(
      pl.kernel,
      out_type=[part, part],
      mesh=mesh,
      scratch_types=[
          pltpu.VMEM((CHUNK,), jnp.int32),
          pltpu.VMEM((CHUNK,), jnp.int32),
          pltpu.VMEM((TAIL,), jnp.int32),
          pltpu.VMEM((CHUNK, D), jnp.float32),
          pltpu.VMEM_SHARED((N_PAD, D), jnp.float32),
          pltpu.SemaphoreType.DMA,
          pltpu.SemaphoreType.DMA,
      ],
      compiler_params=_sc_compiler_params(),
  )
  def sc_degree(ei_hbm, out0, out1, didx_a, didx_b, didx_t, ones_rows, acc,
                ssem_a, ssem_b):
    c = lax.axis_index("c")
    s = lax.axis_index("s")
    wid = s * NC + c
    rbase = s * R_T
    _fill_rows(ones_rows, CHUNK, 0.0)
    _zero_acc_slab(ones_rows, acc, rbase, CHUNK)
    _fill_rows(ones_rows, CHUNK, 1.0)
    plsc.subcore_barrier()

    ebase = wid * T_E

    def half(eoff, didx, ssem, first):
      if not first:
        pltpu.make_async_copy(ones_rows, acc.at[didx], ssem).wait()
      pltpu.sync_copy(ei_hbm.at[1, pl.ds(eoff, CHUNK)], didx)
      pltpu.async_copy(ones_rows, acc.at[didx], ssem, add=True)

    def pair(p, first):
      eoff = ebase + p * 2 * CHUNK
      half(eoff, didx_a, ssem_a, first)
      half(eoff + CHUNK, didx_b, ssem_b, first)

    pair(0, True)

    @pl.loop(1, N_PAIRS)
    def _(p):
      pair(p, False)

    eoff = ebase + N_PAIRS * 2 * CHUNK
    half(eoff, didx_a, ssem_a, False)
    pltpu.sync_copy(ei_hbm.at[1, pl.ds(eoff + CHUNK, TAIL)], didx_t)
    ones_t = ones_rows.at[pl.ds(0, TAIL)]
    pltpu.make_async_copy(ones_rows, acc.at[didx_b], ssem_b).wait()
    pltpu.async_copy(ones_t, acc.at[didx_t], ssem_b, add=True)

    pltpu.make_async_copy(ones_rows, acc.at[didx_a], ssem_a).wait()
    pltpu.make_async_copy(ones_t, acc.at[didx_t], ssem_b).wait()

    plsc.subcore_barrier()
    _dump_slab(acc, rbase, c, out0, out1)

  return sc_degree


def _tc_call(body, n_in, n_out):
    spec = pl.BlockSpec((TC_BLK, 128), lambda i: (i, 0))
    return pl.pallas_call(
        body,
        grid=(F // TC_BLK,),
        in_specs=[spec] * n_in,
        out_specs=[spec] * n_out if n_out > 1 else spec,
        out_shape=(
            [jax.ShapeDtypeStruct((F, 128), jnp.float32)] * n_out
            if n_out > 1 else jax.ShapeDtypeStruct((F, 128), jnp.float32)),
    )


def _tc_init_body(d0_ref, d1_ref, x0_ref, dinv_ref, z0_ref):
    deg = d0_ref[...] + d1_ref[...]
    dinv = jnp.where(deg > 0, lax.rsqrt(jnp.maximum(deg, 1e-12)),
                     jnp.float32(0.0))
    dinv_ref[...] = dinv
    z0_ref[...] = x0_ref[...] * dinv


def _tc_combine_body(p0_ref, p1_ref, dinv_ref, x_ref, z_ref):
    dinv = dinv_ref[...]
    x = dinv * (p0_ref[...] + p1_ref[...])
    x_ref[...] = x
    z_ref[...] = x * dinv


def _tc_final_body(p0_ref, p1_ref, dinv_ref, x0_ref, x1_ref, x2_ref, out_ref):
    x3 = dinv_ref[...] * (p0_ref[...] + p1_ref[...])
    out_ref[...] = (x0_ref[...] + x1_ref[...] + x2_ref[...] + x3) * 0.25


def _flat(a):
    return a.reshape(F, 128)


def kernel(user_emb, item_emb, edge_index):
    ei = edge_index.astype(jnp.int32)

    fu = NUM_USERS * D // 128
    x0f = jnp.concatenate(
        [user_emb.reshape(fu, 128), item_emb.reshape(fu, 128),
         jnp.zeros((F - 2 * fu, 128), jnp.float32)], axis=0)

    sc_segment_sum = _make_sc_segment_sum()
    dg0, dg1 = _make_sc_degree()(ei)
    dinvf, zf = _tc_call(_tc_init_body, 3, 2)(_flat(dg0), _flat(dg1), x0f)

    xfs = []
    for _ in range(N_LAYERS - 1):
        p0, p1 = sc_segment_sum(zf.reshape(N_PAD, D), ei)
        xf, zf = _tc_call(_tc_combine_body, 3, 2)(_flat(p0), _flat(p1), dinvf)
        xfs.append(xf)
    p0, p1 = sc_segment_sum(zf.reshape(N_PAD, D), ei)
    outf = _tc_call(_tc_final_body, 6, 1)(
        _flat(p0), _flat(p1), dinvf, x0f, xfs[0], xfs[1])

    fu = NUM_USERS * D // 128
    users = outf[:fu].reshape(NUM_USERS, D)
    items = outf[fu:2 * fu].reshape(NUM_ITEMS, D)
    return (users, items)


# speedup vs baseline: 69.0926x; 1.2060x over previous
import functools

import jax
import jax.numpy as jnp
from jax import lax
from jax.experimental import pallas as pl
from jax.experimental.pallas import tpu as pltpu
from jax.experimental.pallas import tpu_sc as plsc

NUM_USERS = 25000
NUM_ITEMS = 25000
N = NUM_USERS + NUM_ITEMS
E = 1600000
D = 32
N_LAYERS = 3

NC, NS = 2, 16
NT = NC * NS
N_PAD = 50176
R_T = N_PAD // NS
CHUNK = 440
T_E = E // NT
N_PAIRS = 56
TAIL = T_E - N_PAIRS * 2 * CHUNK - CHUNK
F = N_PAD * D // 128
TC_BLK = 896


def _sc_compiler_params():
  return pltpu.CompilerParams(use_tc_tiling_on_sc=False)


def _fill_rows(rows_ref, n_rows, value):
  vec = jnp.full((16,), value, jnp.float32)

  @pl.loop(0, n_rows)
  def _(r):
    rows_ref[r, pl.ds(0, 16)] = vec
    rows_ref[r, pl.ds(16, 16)] = vec


def _zero_acc_slab(rows_ref, acc, rbase, buf_rows):
  n_full = R_T // buf_rows
  rem = R_T - n_full * buf_rows

  @pl.loop(0, n_full)
  def _(i):
    pltpu.sync_copy(rows_ref, acc.at[pl.ds(rbase + i * buf_rows, buf_rows)])

  if rem:
    pltpu.sync_copy(rows_ref.at[pl.ds(0, rem)],
                    acc.at[pl.ds(rbase + n_full * buf_rows, rem)])


def _dump_slab(acc, rbase, c, out0, out1):
  @pl.when(c == 0)
  def _():
    pltpu.sync_copy(acc.at[pl.ds(rbase, R_T)], out0.at[pl.ds(rbase, R_T)])

  @pl.when(c == 1)
  def _():
    pltpu.sync_copy(acc.at[pl.ds(rbase, R_T)], out1.at[pl.ds(rbase, R_T)])


@functools.cache
def _make_sc_segment_sum():
  mesh = plsc.VectorSubcoreMesh(
      core_axis_name="c", subcore_axis_name="s", num_cores=NC, num_subcores=NS)
  part = jax.ShapeDtypeStruct((N_PAD, D), jnp.float32)

  @functools.partial(
      pl.kernel,
      out_type=[part, part],
      mesh=mesh,
      scratch_types=[
          pltpu.VMEM((2, CHUNK), jnp.int32),
          pltpu.VMEM((2, CHUNK), jnp.int32),
          pltpu.VMEM((2, TAIL), jnp.int32),
          pltpu.VMEM((CHUNK, D), jnp.float32),
          pltpu.VMEM((CHUNK, D), jnp.float32),
          pltpu.VMEM_SHARED((N_PAD, D), jnp.float32),
          pltpu.SemaphoreType.DMA,
          pltpu.SemaphoreType.DMA,
          pltpu.SemaphoreType.DMA,
      ],
      compiler_params=_sc_compiler_params(),
  )
  def sc_segment_sum(z_hbm, ei_hbm, out0, out1,
                     idx_a, idx_b, idx_t,
                     rows_a, rows_b, acc, gsem, ssem_a, ssem_b):
    c = lax.axis_index("c")
    s = lax.axis_index("s")
    wid = s * NC + c
    rbase = s * R_T
    _fill_rows(rows_a, CHUNK, 0.0)
    _zero_acc_slab(rows_a, acc, rbase, CHUNK)
    plsc.subcore_barrier()

    ebase = wid * T_E

    def scatter_desc(rows, idx, ssem):
      return pltpu.make_async_copy(rows, acc.at[idx.at[1]], ssem)

    def load_idx(eoff, n, idx):
      pltpu.sync_copy(ei_hbm.at[pl.ds(0, 2), pl.ds(eoff, n)], idx)

    def gather_split(idx, rows):
      h1 = 224
      g1 = pltpu.async_copy(z_hbm.at[idx.at[0, pl.ds(0, h1)]],
                            rows.at[pl.ds(0, h1)], gsem)
      g2 = pltpu.async_copy(z_hbm.at[idx.at[0, pl.ds(h1, CHUNK - h1)]],
                            rows.at[pl.ds(h1, CHUNK - h1)], gsem)
      return (g1, g2)

    def pair(p, first):
      eoff = ebase + p * 2 * CHUNK
      if not first:
        scatter_desc(rows_a, idx_a, ssem_a).wait()
      load_idx(eoff, CHUNK, idx_a)
      ga = gather_split(idx_a, rows_a)
      if not first:
        scatter_desc(rows_b, idx_b, ssem_b).wait()
      load_idx(eoff + CHUNK, CHUNK, idx_b)
      gb = gather_split(idx_b, rows_b)
      for g in ga:
        g.wait()
      pltpu.async_copy(rows_a, acc.at[idx_a.at[1]], ssem_a, add=True)
      for g in gb:
        g.wait()
      pltpu.async_copy(rows_b, acc.at[idx_b.at[1]], ssem_b, add=True)

    pair(0, True)

    @pl.loop(1, N_PAIRS)
    def _(p):
      pair(p, False)

    eoff = ebase + N_PAIRS * 2 * CHUNK
    scatter_desc(rows_a, idx_a, ssem_a).wait()
    load_idx(eoff, CHUNK, idx_a)
    ga = gather_split(idx_a, rows_a)
    scatter_desc(rows_b, idx_b, ssem_b).wait()
    load_idx(eoff + CHUNK, TAIL, idx_t)
    rows_t = rows_b.at[pl.ds(0, TAIL)]
    gb = pltpu.async_copy(z_hbm.at[idx_t.at[0]], rows_t, gsem)
    for g in ga:
      g.wait()
    pltpu.async_copy(rows_a, acc.at[idx_a.at[1]], ssem_a, add=True)
    gb.wait()
    pltpu.async_copy(rows_t, acc.at[idx_t.at[1]], ssem_b, add=True)
    scatter_desc(rows_a, idx_a, ssem_a).wait()
    pltpu.make_async_copy(rows_t, acc.at[idx_t.at[1]], ssem_b).wait()

    plsc.subcore_barrier()
    _dump_slab(acc, rbase, c, out0, out1)

  return sc_segment_sum


@functools.cache
def _make_sc_degree():
  mesh = plsc.VectorSubcoreMesh(
      core_axis_name="c", subcore_axis_name="s", num_cores=NC, num_subcores=NS)
  part = jax.ShapeDtypeStruct((N_PAD, D), jnp.float32)

  @functools.partial(
      pl.kernel,
      out_type=[part, part],
      mesh=mesh,
      scratch_types=[
          [pltpu.VMEM((CHUNK,), jnp.int32)] * 4,
          pltpu.VMEM((TAIL,), jnp.int32),
          pltpu.VMEM((CHUNK, D), jnp.float32),
          pltpu.VMEM_SHARED((N_PAD, D), jnp.float32),
          [pltpu.SemaphoreType.DMA] * 4,
      ],
      compiler_params=_sc_compiler_params(),
  )
  def sc_degree(ei_hbm, out0, out1, didxs, didx_t, ones_rows, acc, ssems):
    c = lax.axis_index("c")
    s = lax.axis_index("s")
    wid = s * NC + c
    rbase = s * R_T
    _fill_rows(ones_rows, CHUNK, 0.0)
    _zero_acc_slab(ones_rows, acc, rbase, CHUNK)
    _fill_rows(ones_rows, CHUNK, 1.0)
    plsc.subcore_barrier()

    ebase = wid * T_E
    NQ = 4
    N_QUADS = T_E // (NQ * CHUNK)
    REST = T_E - N_QUADS * NQ * CHUNK - TAIL

    def half(eoff, didx, ssem, first):
      if not first:
        pltpu.make_async_copy(ones_rows, acc.at[didx], ssem).wait()
      pltpu.sync_copy(ei_hbm.at[1, pl.ds(eoff, CHUNK)], didx)
      pltpu.async_copy(ones_rows, acc.at[didx], ssem, add=True)

    def quad(q, first):
      eoff = ebase + q * NQ * CHUNK
      for j in range(NQ):
        half(eoff + j * CHUNK, didxs[j], ssems[j], first)

    quad(0, True)

    @pl.loop(1, N_QUADS)
    def _(q):
      quad(q, False)

    eoff = ebase + N_QUADS * NQ * CHUNK
    n_rest = REST // CHUNK
    for j in range(n_rest):
      half(eoff + j * CHUNK, didxs[j], ssems[j], False)
    pltpu.sync_copy(ei_hbm.at[1, pl.ds(eoff + n_rest * CHUNK, TAIL)], didx_t)
    ones_t = ones_rows.at[pl.ds(0, TAIL)]
    pltpu.make_async_copy(ones_rows, acc.at[didxs[n_rest]],
                          ssems[n_rest]).wait()
    pltpu.async_copy(ones_t, acc.at[didx_t], ssems[n_rest], add=True)

    for j in range(NQ):
      if j == n_rest:
        pltpu.make_async_copy(ones_t, acc.at[didx_t], ssems[j]).wait()
      else:
        pltpu.make_async_copy(ones_rows, acc.at[didxs[j]], ssems[j]).wait()

    plsc.subcore_barrier()
    _dump_slab(acc, rbase, c, out0, out1)

  return sc_degree


def _tc_call(body, n_in, n_out):
    spec = pl.BlockSpec((TC_BLK, 128), lambda i: (i, 0))
    return pl.pallas_call(
        body,
        grid=(F // TC_BLK,),
        in_specs=[spec] * n_in,
        out_specs=[spec] * n_out if n_out > 1 else spec,
        out_shape=(
            [jax.ShapeDtypeStruct((F, 128), jnp.float32)] * n_out
            if n_out > 1 else jax.ShapeDtypeStruct((F, 128), jnp.float32)),
    )


def _tc_init_body(d0_ref, d1_ref, x0_ref, dinv_ref, z0_ref):
    deg = d0_ref[...] + d1_ref[...]
    dinv = jnp.where(deg > 0, lax.rsqrt(jnp.maximum(deg, 1e-12)),
                     jnp.float32(0.0))
    dinv_ref[...] = dinv
    z0_ref[...] = x0_ref[...] * dinv


def _tc_combine_body(p0_ref, p1_ref, dinv_ref, x_ref, z_ref):
    dinv = dinv_ref[...]
    x = dinv * (p0_ref[...] + p1_ref[...])
    x_ref[...] = x
    z_ref[...] = x * dinv


def _tc_final_body(p0_ref, p1_ref, dinv_ref, x0_ref, x1_ref, x2_ref, out_ref):
    x3 = dinv_ref[...] * (p0_ref[...] + p1_ref[...])
    out_ref[...] = (x0_ref[...] + x1_ref[...] + x2_ref[...] + x3) * 0.25


def _flat(a):
    return a.reshape(F, 128)


def kernel(user_emb, item_emb, edge_index):
    ei = edge_index.astype(jnp.int32)

    fu = NUM_USERS * D // 128
    x0f = jnp.concatenate(
        [user_emb.reshape(fu, 128), item_emb.reshape(fu, 128),
         jnp.zeros((F - 2 * fu, 128), jnp.float32)], axis=0)

    sc_segment_sum = _make_sc_segment_sum()
    dg0, dg1 = _make_sc_degree()(ei)
    dinvf, zf = _tc_call(_tc_init_body, 3, 2)(_flat(dg0), _flat(dg1), x0f)

    xfs = []
    for _ in range(N_LAYERS - 1):
        p0, p1 = sc_segment_sum(zf.reshape(N_PAD, D), ei)
        xf, zf = _tc_call(_tc_combine_body, 3, 2)(_flat(p0), _flat(p1), dinvf)
        xfs.append(xf)
    p0, p1 = sc_segment_sum(zf.reshape(N_PAD, D), ei)
    outf = _tc_call(_tc_final_body, 6, 1)(
        _flat(p0), _flat(p1), dinvf, x0f, xfs[0], xfs[1])

    fu = NUM_USERS * D // 128
    users = outf[:fu].reshape(NUM_USERS, D)
    items = outf[fu:2 * fu].reshape(NUM_ITEMS, D)
    return (users, items)
